# baseline (speedup 1.0000x reference)
import functools

import jax
import jax.numpy as jnp
from jax import lax
from jax.experimental import pallas as pl
from jax.experimental.pallas import tpu as pltpu
from jax.experimental.pallas import tpu_sc as plsc


def _sc_row_gather(x, index, num_rows, d):
    nw = 16
    b_per_w = num_rows // nw
    chunk = 2 * b_per_w
    mesh = plsc.VectorSubcoreMesh(
        core_axis_name="c", subcore_axis_name="s", num_cores=1
    )

    @functools.partial(
        pl.kernel,
        mesh=mesh,
        out_type=jax.ShapeDtypeStruct((num_rows, d), jnp.float32),
        scratch_types=[
            pltpu.VMEM((chunk,), jnp.int32),
            pltpu.VMEM((chunk, d), jnp.float32),
            pltpu.SemaphoreType.DMA,
        ],
    )
    def gather_kernel(x_hbm, idx_hbm, out_hbm, idx_v, rows_v, sem):
        wid = lax.axis_index("s")
        pltpu.sync_copy(idx_hbm.at[pl.ds((wid // 2) * chunk, chunk)], idx_v)
        pltpu.async_copy(x_hbm.at[idx_v], rows_v, sem).wait()
        half = (wid % 2) * b_per_w
        pltpu.sync_copy(
            rows_v.at[pl.ds(half, b_per_w)],
            out_hbm.at[pl.ds(wid * b_per_w, b_per_w)],
        )

    return gather_kernel(x, index)


def kernel(x, index):
    return _sc_row_gather(x, index, index.shape[0], x.shape[1])
